# Initial kernel scaffold; baseline (speedup 1.0000x reference)
import jax
import jax.numpy as jnp
from jax.experimental import pallas as pl


def kernel(hidden_states, router_w, ws, w2s):
    raise NotImplementedError("write your pallas kernel here")


# speedup vs baseline: 1.2709x; 1.2709x over previous
import functools

import jax
import jax.numpy as jnp
from jax import lax
from jax.experimental import pallas as pl
from jax.experimental.pallas import tpu as pltpu
from jax.experimental.pallas import tpu_sc as plsc

T = 2048
DM = 1024
FF = 4096
NE = 8
TM = 256
BF = 1024
NF = FF // BF
NENT = 2 * T
NB = NENT // TM + NE
NPAD = NB * TM

_NTILES = 32
_GROWS = NPAD // _NTILES
_GCH = 64
_CTOK = T // _NTILES
_CCH = 32


def _routing_body(x_ref, rwt_ref, dest_ref, went_ref, be_ref):
    x = x_ref[...]
    logits = jnp.dot(x, rwt_ref[...], preferred_element_type=jnp.float32)
    iota_e = lax.broadcasted_iota(jnp.int32, (T, NE), 1)
    m1 = jnp.max(logits, axis=1, keepdims=True)
    idx1 = jnp.min(jnp.where(logits == m1, iota_e, NE), axis=1, keepdims=True)
    l2 = jnp.where(iota_e == idx1, -jnp.inf, logits)
    m2 = jnp.max(l2, axis=1, keepdims=True)
    idx2 = jnp.min(jnp.where(l2 == m2, iota_e, NE), axis=1, keepdims=True)
    w1 = 1.0 / (1.0 + jnp.exp(m2 - m1))
    w2 = 1.0 - w1

    e_flat = jnp.concatenate([idx1, idx2], axis=0)
    w_flat = jnp.concatenate([w1, w2], axis=0)
    oh = (lax.broadcasted_iota(jnp.int32, (NENT, NE), 1) == e_flat).astype(
        jnp.float32)

    CB = 512
    r_i = lax.broadcasted_iota(jnp.int32, (CB, CB), 0)
    c_i = lax.broadcasted_iota(jnp.int32, (CB, CB), 1)
    stril = (r_i > c_i).astype(jnp.float32)
    carry = jnp.zeros((1, NE), jnp.float32)
    ranks = []
    for b in range(NENT // CB):
        blk = oh[b * CB:(b + 1) * CB]
        ranks.append(jnp.dot(stril, blk, preferred_element_type=jnp.float32)
                     + carry)
        carry = carry + jnp.sum(blk, axis=0, keepdims=True)
    rank = jnp.concatenate(ranks, axis=0)

    counts = carry
    pc = jnp.ceil(counts / TM) * TM
    e_r = lax.broadcasted_iota(jnp.int32, (NE, NE), 0)
    e_c = lax.broadcasted_iota(jnp.int32, (NE, NE), 1)
    striu = (e_r < e_c).astype(jnp.float32)
    offs = jnp.dot(pc, striu, preferred_element_type=jnp.float32)

    dest = jnp.sum(oh * (rank + offs), axis=1, keepdims=True)
    dest_ref[...] = dest.astype(jnp.int32)
    went_ref[...] = w_flat

    pos = lax.broadcasted_iota(jnp.int32, (NB, 1), 0).astype(jnp.float32) * TM
    be_ref[...] = jnp.sum((pos >= offs).astype(jnp.int32), axis=1,
                          keepdims=True) - 1


_routing = pl.pallas_call(
    _routing_body,
    out_shape=(
        jax.ShapeDtypeStruct((NENT, 1), jnp.int32),
        jax.ShapeDtypeStruct((NENT, 1), jnp.float32),
        jax.ShapeDtypeStruct((NB, 1), jnp.int32),
    ),
)


def _dispatch_build_body(dest_hbm, went_hbm, tid_hbm, wsort_hbm,
                         dest_v, went_v, tid_v, ws_v):
    cid = lax.axis_index("c")
    sid = lax.axis_index("s")

    @pl.when(jnp.logical_and(cid == 0, sid == 0))
    def _():
        pltpu.sync_copy(dest_hbm, dest_v)
        pltpu.sync_copy(went_hbm, went_v)

        def init(i, _):
            tid_v[pl.ds(i * 16, 16)] = jnp.zeros((16,), jnp.int32)
            ws_v[pl.ds(i * 16, 16)] = jnp.zeros((16,), jnp.float32)
            return 0

        lax.fori_loop(0, NPAD // 16, init, 0)

        def scatter(i, _):
            d = dest_v[pl.ds(i * 16, 16)]
            ids = lax.iota(jnp.int32, 16) + i * 16
            tid = jnp.where(ids >= T, ids - T, ids)
            w = went_v[pl.ds(i * 16, 16)]
            plsc.store_scatter(tid_v, [d], tid)
            plsc.store_scatter(ws_v, [d], w)
            return 0

        lax.fori_loop(0, NENT // 16, scatter, 0)
        pltpu.sync_copy(tid_v, tid_hbm)
        pltpu.sync_copy(ws_v, wsort_hbm)


def _gather_rows_body(x_hbm, tid_hbm, xs_hbm, idx_v, rows_v, sem):
    wid = lax.axis_index("s") * 2 + lax.axis_index("c")
    base = wid * _GROWS

    def chunk(i, _):
        b = base + i * _GCH
        pltpu.sync_copy(tid_hbm.at[pl.ds(b, _GCH)], idx_v)
        pltpu.async_copy(x_hbm.at[idx_v], rows_v, sem).wait()
        pltpu.sync_copy(rows_v, xs_hbm.at[pl.ds(b, _GCH)])
        return 0

    lax.fori_loop(0, _GROWS // _GCH, chunk, 0)


def _mlp_body(be_ref, x_ref, g_ref, u_ref, w2_ref, w_ref, out_ref, acc_ref):
    f = pl.program_id(0)
    r = pl.program_id(1)
    x = x_ref[...]
    dn = (((1,), (1,)), ((), ()))
    g = lax.dot_general(x, g_ref[0], dn,
                        preferred_element_type=jnp.float32)
    u = lax.dot_general(x, u_ref[0], dn,
                        preferred_element_type=jnp.float32)
    h = g * (1.0 / (1.0 + jnp.exp(-g))) * u
    part = lax.dot_general(h, w2_ref[0], dn,
                           preferred_element_type=jnp.float32)
    sl = pl.ds(r * TM, TM)

    @pl.when(f == 0)
    def _():
        acc_ref[sl, :] = part

    @pl.when(f > 0)
    def _():
        acc_ref[sl, :] = acc_ref[sl, :] + part

    out_ref[...] = acc_ref[sl, :] * w_ref[...]


_mlp = pl.pallas_call(
    _mlp_body,
    grid_spec=pltpu.PrefetchScalarGridSpec(
        num_scalar_prefetch=1,
        grid=(NF, NB),
        in_specs=[
            pl.BlockSpec((TM, DM), lambda f, r, be: (r, 0)),
            pl.BlockSpec((1, BF, DM), lambda f, r, be: (be[r], f, 0)),
            pl.BlockSpec((1, BF, DM), lambda f, r, be: (be[r], NF + f, 0)),
            pl.BlockSpec((1, DM, BF), lambda f, r, be: (be[r], 0, f)),
            pl.BlockSpec((TM, 1), lambda f, r, be: (r, 0)),
        ],
        out_specs=pl.BlockSpec((TM, DM), lambda f, r, be: (r, 0)),
        scratch_shapes=[pltpu.VMEM((NPAD, DM), jnp.float32)],
    ),
    out_shape=jax.ShapeDtypeStruct((NPAD, DM), jnp.float32),
    compiler_params=pltpu.CompilerParams(
        dimension_semantics=("arbitrary", "arbitrary")),
)


def _combine_body(outs_hbm, dest_hbm, out_hbm, i0_v, i1_v, a_v, b_v, sem):
    wid = lax.axis_index("s") * 2 + lax.axis_index("c")
    base = wid * _CTOK

    def chunk(i, _):
        b = base + i * _CCH
        pltpu.sync_copy(dest_hbm.at[pl.ds(b, _CCH)], i0_v)
        pltpu.sync_copy(dest_hbm.at[pl.ds(T + b, _CCH)], i1_v)
        c0 = pltpu.async_copy(outs_hbm.at[i0_v], a_v, sem)
        c1 = pltpu.async_copy(outs_hbm.at[i1_v], b_v, sem)
        c0.wait()
        c1.wait()

        def add(j, _):
            row = j // (DM // 16)
            col = (j % (DM // 16)) * 16
            a_v[row, pl.ds(col, 16)] = (a_v[row, pl.ds(col, 16)]
                                        + b_v[row, pl.ds(col, 16)])
            return 0

        lax.fori_loop(0, _CCH * DM // 16, add, 0)
        pltpu.sync_copy(a_v, out_hbm.at[pl.ds(b, _CCH)])
        return 0

    lax.fori_loop(0, _CTOK // _CCH, chunk, 0)


@functools.lru_cache(maxsize=1)
def _sc_kernels():
    mesh = plsc.VectorSubcoreMesh(core_axis_name="c", subcore_axis_name="s")
    sc_params = pltpu.CompilerParams(needs_layout_passes=False)
    dispatch = pl.kernel(
        _dispatch_build_body,
        out_type=(jax.ShapeDtypeStruct((NPAD,), jnp.int32),
                  jax.ShapeDtypeStruct((NPAD,), jnp.float32)),
        mesh=mesh,
        scratch_types=[
            pltpu.VMEM((NENT,), jnp.int32),
            pltpu.VMEM((NENT,), jnp.float32),
            pltpu.VMEM((NPAD,), jnp.int32),
            pltpu.VMEM((NPAD,), jnp.float32),
        ],
        compiler_params=sc_params,
    )
    gather = pl.kernel(
        _gather_rows_body,
        out_type=jax.ShapeDtypeStruct((NPAD, DM), jnp.float32),
        mesh=mesh,
        scratch_types=[
            pltpu.VMEM((_GCH,), jnp.int32),
            pltpu.VMEM((_GCH, DM), jnp.float32),
            pltpu.SemaphoreType.DMA,
        ],
        compiler_params=sc_params,
    )
    combine = pl.kernel(
        _combine_body,
        out_type=jax.ShapeDtypeStruct((T, DM), jnp.float32),
        mesh=mesh,
        scratch_types=[
            pltpu.VMEM((_CCH,), jnp.int32),
            pltpu.VMEM((_CCH,), jnp.int32),
            pltpu.VMEM((_CCH, DM), jnp.float32),
            pltpu.VMEM((_CCH, DM), jnp.float32),
            pltpu.SemaphoreType.DMA,
        ],
        compiler_params=sc_params,
    )
    return dispatch, gather, combine


@jax.jit
def kernel(hidden_states, router_w, ws, w2s):
    dispatch, gather, combine = _sc_kernels()
    x = hidden_states.reshape(T, DM)
    dest, went, be = _routing(x, router_w.T)
    dest_f = dest.reshape(NENT)
    tid, wsort = dispatch(dest_f, went.reshape(NENT))
    xs = gather(x, tid)
    outs = _mlp(be.reshape(NB), xs, ws, ws, w2s, wsort.reshape(NPAD, 1))
    return combine(outs, dest_f)


# speedup vs baseline: 1.6914x; 1.3309x over previous
import functools

import jax
import jax.numpy as jnp
from jax import lax
from jax.experimental import pallas as pl
from jax.experimental.pallas import tpu as pltpu
from jax.experimental.pallas import tpu_sc as plsc

T = 2048
DM = 1024
FF = 4096
NE = 8
TM = 256
BF = 1024
NF = FF // BF
NENT = 2 * T
NB = NENT // TM + NE
NPAD = NB * TM

_NTILES = 32
_GROWS = NPAD // _NTILES
_GCH = 48
_CTOK = T // _NTILES
_CCH = 16


def _routing_body(x_ref, rwt_ref, dest_ref, went_ref, offs_ref, nblk_ref):
    x = x_ref[...]
    logits = jnp.dot(x, rwt_ref[...], preferred_element_type=jnp.float32)
    iota_e = lax.broadcasted_iota(jnp.int32, (T, NE), 1)
    m1 = jnp.max(logits, axis=1, keepdims=True)
    idx1 = jnp.min(jnp.where(logits == m1, iota_e, NE), axis=1, keepdims=True)
    l2 = jnp.where(iota_e == idx1, -jnp.inf, logits)
    m2 = jnp.max(l2, axis=1, keepdims=True)
    idx2 = jnp.min(jnp.where(l2 == m2, iota_e, NE), axis=1, keepdims=True)
    w1 = 1.0 / (1.0 + jnp.exp(m2 - m1))
    w2 = 1.0 - w1

    e_flat = jnp.concatenate([idx1, idx2], axis=0)
    w_flat = jnp.concatenate([w1, w2], axis=0)
    oh = (lax.broadcasted_iota(jnp.int32, (NENT, NE), 1) == e_flat).astype(
        jnp.float32)

    CB = 512
    r_i = lax.broadcasted_iota(jnp.int32, (CB, CB), 0)
    c_i = lax.broadcasted_iota(jnp.int32, (CB, CB), 1)
    stril = (r_i > c_i).astype(jnp.float32)
    carry = jnp.zeros((1, NE), jnp.float32)
    ranks = []
    for b in range(NENT // CB):
        blk = oh[b * CB:(b + 1) * CB]
        ranks.append(jnp.dot(stril, blk, preferred_element_type=jnp.float32)
                     + carry)
        carry = carry + jnp.sum(blk, axis=0, keepdims=True)
    rank = jnp.concatenate(ranks, axis=0)

    counts = carry
    pc = jnp.ceil(counts / TM) * TM
    e_r = lax.broadcasted_iota(jnp.int32, (NE, NE), 0)
    e_c = lax.broadcasted_iota(jnp.int32, (NE, NE), 1)
    striu = (e_r < e_c).astype(jnp.float32)
    offs = jnp.dot(pc, striu, preferred_element_type=jnp.float32)

    dest = jnp.sum(oh * (rank + offs), axis=1, keepdims=True)
    dest_ref[...] = dest.astype(jnp.int32)
    went_ref[...] = w_flat
    offs_ref[...] = offs.astype(jnp.int32)
    nblk_ref[...] = (pc / TM).astype(jnp.int32)


_routing = pl.pallas_call(
    _routing_body,
    out_shape=(
        jax.ShapeDtypeStruct((NENT, 1), jnp.int32),
        jax.ShapeDtypeStruct((NENT, 1), jnp.float32),
        jax.ShapeDtypeStruct((1, NE), jnp.int32),
        jax.ShapeDtypeStruct((1, NE), jnp.int32),
    ),
)


def _dispatch_build_body(dest_hbm, went_hbm, tid_hbm, wsort_hbm,
                         dest_v, went_v, tid_v, ws_v):
    cid = lax.axis_index("c")
    sid = lax.axis_index("s")

    @pl.when(jnp.logical_and(cid == 0, sid == 0))
    def _():
        pltpu.sync_copy(dest_hbm, dest_v)
        pltpu.sync_copy(went_hbm, went_v)

        def init(i, _):
            tid_v[pl.ds(i * 16, 16)] = jnp.zeros((16,), jnp.int32)
            ws_v[pl.ds(i * 16, 16)] = jnp.zeros((16,), jnp.float32)
            return 0

        lax.fori_loop(0, NPAD // 16, init, 0)

        def scatter(i, _):
            d = dest_v[pl.ds(i * 16, 16)]
            ids = lax.iota(jnp.int32, 16) + i * 16
            tid = jnp.where(ids >= T, ids - T, ids)
            w = went_v[pl.ds(i * 16, 16)]
            plsc.store_scatter(tid_v, [d], tid)
            plsc.store_scatter(ws_v, [d], w)
            return 0

        lax.fori_loop(0, NENT // 16, scatter, 0)
        pltpu.sync_copy(tid_v, tid_hbm)
        pltpu.sync_copy(ws_v, wsort_hbm)


def _gather_rows_body(x_hbm, tid_hbm, xs_hbm, idx_v, rA, rB, gA, gB, sA, sB):
    wid = lax.axis_index("s") * 2 + lax.axis_index("c")
    base = wid * _GROWS
    pltpu.sync_copy(tid_hbm.at[pl.ds(base, _GROWS)], idx_v)
    nch = _GROWS // _GCH
    bufs = (rA, rB)
    gsem = (gA, gB)
    ssem = (sA, sB)
    pltpu.async_copy(x_hbm.at[idx_v.at[pl.ds(0, _GCH)]], bufs[0], gsem[0])
    for i in range(nch):
        if i + 1 < nch:
            if i >= 1:
                pltpu.make_async_copy(bufs[(i + 1) % 2],
                                      xs_hbm.at[pl.ds(base + (i - 1) * _GCH,
                                                      _GCH)],
                                      ssem[(i + 1) % 2]).wait()
            pltpu.async_copy(x_hbm.at[idx_v.at[pl.ds((i + 1) * _GCH, _GCH)]],
                             bufs[(i + 1) % 2], gsem[(i + 1) % 2])
        pltpu.make_async_copy(x_hbm.at[idx_v.at[pl.ds(i * _GCH, _GCH)]],
                              bufs[i % 2], gsem[i % 2]).wait()
        pltpu.async_copy(bufs[i % 2], xs_hbm.at[pl.ds(base + i * _GCH, _GCH)],
                         ssem[i % 2])
    pltpu.make_async_copy(bufs[(nch - 2) % 2],
                          xs_hbm.at[pl.ds(base + (nch - 2) * _GCH, _GCH)],
                          ssem[(nch - 2) % 2]).wait()
    pltpu.make_async_copy(bufs[(nch - 1) % 2],
                          xs_hbm.at[pl.ds(base + (nch - 1) * _GCH, _GCH)],
                          ssem[(nch - 1) % 2]).wait()


def _mlp_body(offs_ref, nblk_ref, x_any, w_any, g_ref, u_ref, w2_ref, out_any,
              xv_ref, wv_ref, acc_ref, sem):
    e = pl.program_id(0)
    f = pl.program_id(1)

    @pl.when(jnp.logical_and(f == 0, e == 0))
    def _():
        cp = pltpu.make_async_copy(x_any, xv_ref, sem)
        cp.start()
        cp.wait()
        cp2 = pltpu.make_async_copy(w_any, wv_ref, sem)
        cp2.start()
        cp2.wait()

    base = offs_ref[e]
    nb = nblk_ref[e]
    dn = (((1,), (1,)), ((), ()))
    prec = lax.Precision.DEFAULT

    def blk(rb, _):
        lsl = pl.ds(pl.multiple_of(rb * TM, TM), TM)
        gsl = pl.ds(pl.multiple_of(base + rb * TM, TM), TM)
        x = xv_ref[gsl, :]
        g = lax.dot_general(x, g_ref[0], dn, precision=prec,
                            preferred_element_type=jnp.float32)
        u = lax.dot_general(x, u_ref[0], dn, precision=prec,
                            preferred_element_type=jnp.float32)
        h = g * (1.0 / (1.0 + jnp.exp(-g))) * u * wv_ref[gsl, :]
        part = lax.dot_general(h, w2_ref[0], dn, precision=prec,
                               preferred_element_type=jnp.float32)

        @pl.when(f == 0)
        def _():
            acc_ref[lsl, :] = part

        @pl.when(f > 0)
        def _():
            acc_ref[lsl, :] = acc_ref[lsl, :] + part

        @pl.when(f == NF - 1)
        def _():
            cp = pltpu.make_async_copy(acc_ref.at[lsl, :], out_any.at[gsl, :],
                                       sem)
            cp.start()
            cp.wait()

        return 0

    lax.fori_loop(0, nb, blk, 0)


_mlp = pl.pallas_call(
    _mlp_body,
    grid_spec=pltpu.PrefetchScalarGridSpec(
        num_scalar_prefetch=2,
        grid=(NE, NF),
        in_specs=[
            pl.BlockSpec(memory_space=pl.ANY),
            pl.BlockSpec(memory_space=pl.ANY),
            pl.BlockSpec((1, BF, DM), lambda e, f, offs, nblk: (e, f, 0)),
            pl.BlockSpec((1, BF, DM), lambda e, f, offs, nblk: (e, NF + f, 0)),
            pl.BlockSpec((1, DM, BF), lambda e, f, offs, nblk: (e, 0, f)),
        ],
        out_specs=pl.BlockSpec(memory_space=pl.ANY),
        scratch_shapes=[
            pltpu.VMEM((NPAD, DM), jnp.float32),
            pltpu.VMEM((NPAD, 1), jnp.float32),
            pltpu.VMEM((T, DM), jnp.float32),
            pltpu.SemaphoreType.DMA,
        ],
    ),
    out_shape=jax.ShapeDtypeStruct((NPAD, DM), jnp.float32),
    compiler_params=pltpu.CompilerParams(
        dimension_semantics=("arbitrary", "arbitrary"),
        vmem_limit_bytes=63 * 1024 * 1024),
)


def _combine_body(outs_hbm, dest_hbm, out_hbm, i0_v, i1_v,
                  a0, b0, a1, b1, g0, g1, s0, s1):
    wid = lax.axis_index("s") * 2 + lax.axis_index("c")
    base = wid * _CTOK
    pltpu.sync_copy(dest_hbm.at[pl.ds(base, _CTOK)], i0_v)
    pltpu.sync_copy(dest_hbm.at[pl.ds(T + base, _CTOK)], i1_v)
    nch = _CTOK // _CCH
    abuf = (a0, a1)
    bbuf = (b0, b1)
    gsem = (g0, g1)
    ssem = (s0, s1)

    def gathers(i):
        s = i % 2
        pltpu.async_copy(outs_hbm.at[i0_v.at[pl.ds(i * _CCH, _CCH)]],
                         abuf[s], gsem[s])
        pltpu.async_copy(outs_hbm.at[i1_v.at[pl.ds(i * _CCH, _CCH)]],
                         bbuf[s], gsem[s])

    def wait_gathers(i):
        s = i % 2
        pltpu.make_async_copy(outs_hbm.at[i0_v.at[pl.ds(i * _CCH, _CCH)]],
                              abuf[s], gsem[s]).wait()
        pltpu.make_async_copy(outs_hbm.at[i1_v.at[pl.ds(i * _CCH, _CCH)]],
                              bbuf[s], gsem[s]).wait()

    def wait_store(i):
        s = i % 2
        pltpu.make_async_copy(abuf[s], out_hbm.at[pl.ds(base + i * _CCH,
                                                        _CCH)],
                              ssem[s]).wait()

    gathers(0)
    for i in range(nch):
        s = i % 2
        if i + 1 < nch:
            if i >= 1:
                wait_store(i - 1)
            gathers(i + 1)
        wait_gathers(i)

        def add(j, _):
            row = j // (DM // 16)
            col = (j % (DM // 16)) * 16
            abuf[s][row, pl.ds(col, 16)] = (abuf[s][row, pl.ds(col, 16)]
                                            + bbuf[s][row, pl.ds(col, 16)])
            return None

        lax.fori_loop(0, _CCH * DM // 16, add, None)
        pltpu.async_copy(abuf[s], out_hbm.at[pl.ds(base + i * _CCH, _CCH)],
                         ssem[s])
    wait_store(nch - 2)
    wait_store(nch - 1)


@functools.lru_cache(maxsize=1)
def _sc_kernels():
    mesh = plsc.VectorSubcoreMesh(core_axis_name="c", subcore_axis_name="s")
    sc_params = pltpu.CompilerParams(needs_layout_passes=False)
    dispatch = pl.kernel(
        _dispatch_build_body,
        out_type=(jax.ShapeDtypeStruct((NPAD,), jnp.int32),
                  jax.ShapeDtypeStruct((NPAD,), jnp.float32)),
        mesh=mesh,
        scratch_types=[
            pltpu.VMEM((NENT,), jnp.int32),
            pltpu.VMEM((NENT,), jnp.float32),
            pltpu.VMEM((NPAD,), jnp.int32),
            pltpu.VMEM((NPAD,), jnp.float32),
        ],
        compiler_params=sc_params,
    )
    gather = pl.kernel(
        _gather_rows_body,
        out_type=jax.ShapeDtypeStruct((NPAD, DM), jnp.float32),
        mesh=mesh,
        scratch_types=[
            pltpu.VMEM((_GROWS,), jnp.int32),
            pltpu.VMEM((_GCH, DM), jnp.float32),
            pltpu.VMEM((_GCH, DM), jnp.float32),
            pltpu.SemaphoreType.DMA,
            pltpu.SemaphoreType.DMA,
            pltpu.SemaphoreType.DMA,
            pltpu.SemaphoreType.DMA,
        ],
        compiler_params=sc_params,
    )
    combine = pl.kernel(
        _combine_body,
        out_type=jax.ShapeDtypeStruct((T, DM), jnp.float32),
        mesh=mesh,
        scratch_types=[
            pltpu.VMEM((_CTOK,), jnp.int32),
            pltpu.VMEM((_CTOK,), jnp.int32),
            pltpu.VMEM((_CCH, DM), jnp.float32),
            pltpu.VMEM((_CCH, DM), jnp.float32),
            pltpu.VMEM((_CCH, DM), jnp.float32),
            pltpu.VMEM((_CCH, DM), jnp.float32),
            pltpu.SemaphoreType.DMA,
            pltpu.SemaphoreType.DMA,
            pltpu.SemaphoreType.DMA,
            pltpu.SemaphoreType.DMA,
        ],
        compiler_params=sc_params,
    )
    return dispatch, gather, combine


@jax.jit
def kernel(hidden_states, router_w, ws, w2s):
    dispatch, gather, combine = _sc_kernels()
    x = hidden_states.reshape(T, DM)
    dest, went, offs, nblk = _routing(x, router_w.T)
    dest_f = dest.reshape(NENT)
    tid, wsort = dispatch(dest_f, went.reshape(NENT))
    xs = gather(x, tid)
    outs = _mlp(offs.reshape(NE), nblk.reshape(NE), xs,
                wsort.reshape(NPAD, 1), ws, ws, w2s)
    return combine(outs, dest_f)
